# baseline (speedup 1.0000x reference)
import jax
import jax.numpy as jnp
from jax import lax
from jax.experimental import pallas as pl
from jax.experimental.pallas import tpu as pltpu
from jax.experimental.pallas import tpu_sc as plsc

A = 26
V = 100000
D = 64
B = 16384
TOTAL = B * A

NC, NS = 2, 16
NW = NC * NS
ROWS_W = TOTAL // NW
IDXW = 128
CHUNK = 512
NJ = CHUNK // IDXW
NCHUNK = ROWS_W // CHUNK


def _body(idx_hbm, tab_hbm, out_hbm, idx_v, half_v, rows_v, out_v, gsem):
    wid = lax.axis_index("s") * NC + lax.axis_index("c")
    lanes = lax.iota(jnp.int32, 16)


    def extract(g, _):
        halves = half_v[pl.ds(g * 16, 16)]
        rows = g * 16 + lanes
        src = halves * 64
        dst = jnp.zeros((16,), jnp.int32)
        for _d in range(D):
            vals = plsc.load_gather(rows_v, [rows, src])
            plsc.store_scatter(out_v, [rows, dst], vals)
            src = src + 1
            dst = dst + 1
        return ()

    def step(c, _):
        base = wid * NCHUNK + c
        pltpu.sync_copy(idx_hbm.at[base], idx_v)
        for q in range(CHUNK // 16):
            v = idx_v[pl.ds(q * 16, 16)]
            half_v[pl.ds(q * 16, 16)] = lax.bitwise_and(v, 1)
            idx_v[pl.ds(q * 16, 16)] = lax.shift_right_logical(v, 1)
        for j in range(NJ):
            pltpu.async_copy(
                tab_hbm.at[idx_v.at[pl.ds(j * IDXW, IDXW)]],
                rows_v.at[pl.ds(j * IDXW, IDXW)],
                gsem,
            )
        for j in range(NJ):
            pltpu.make_async_copy(
                tab_hbm.at[idx_v.at[pl.ds(j * IDXW, IDXW)]],
                rows_v.at[pl.ds(j * IDXW, IDXW)],
                gsem,
            ).wait()
        lax.fori_loop(0, CHUNK // 16, extract, ())
        pltpu.sync_copy(out_v, out_hbm.at[base])
        return ()

    lax.fori_loop(0, NCHUNK, step, ())


def _gather(flat_idx, pair_tab):
    mesh = plsc.VectorSubcoreMesh(core_axis_name="c", subcore_axis_name="s")
    f = pl.kernel(
        _body,
        out_type=jax.ShapeDtypeStruct((TOTAL // CHUNK, CHUNK, D), jnp.float32),
        mesh=mesh,
        scratch_types=[
            pltpu.VMEM((CHUNK,), jnp.int32),
            pltpu.VMEM((CHUNK,), jnp.int32),
            pltpu.VMEM((CHUNK, 128), jnp.float32),
            pltpu.VMEM((CHUNK, D), jnp.float32),
            pltpu.SemaphoreType.DMA,
        ],
        compiler_params=pltpu.CompilerParams(
            use_tc_tiling_on_sc=False, needs_layout_passes=False
        ),
    )
    return f(flat_idx, pair_tab)


def kernel(tuples, tables):
    offs = (jnp.arange(A, dtype=jnp.int32) * V)[None, :]
    flat_idx = (tuples + offs).reshape(TOTAL // CHUNK, CHUNK)
    pair_tab = tables.reshape(A * V // 2, 2 * D)
    out = _gather(flat_idx, pair_tab)
    return out.reshape(B, A * D)


# speedup vs baseline: 1.6477x; 1.6477x over previous
import jax
import jax.numpy as jnp
from jax import lax
from jax.experimental import pallas as pl
from jax.experimental.pallas import tpu as pltpu
from jax.experimental.pallas import tpu_sc as plsc

A = 26
V = 100000
D = 64
B = 16384
TOTAL = B * A

NC, NS = 2, 16
NW = NC * NS
ROWS_W = TOTAL // NW
IDXW = 128
CHUNK = 512
NJ = CHUNK // IDXW
NCHUNK = ROWS_W // CHUNK


def _body(idx_hbm, tab_hbm, out_hbm, idx_v, rows_v, gsem):
    wid = lax.axis_index("s") * NC + lax.axis_index("c")

    def step(c, _):
        base = wid * NCHUNK + c
        pltpu.sync_copy(idx_hbm.at[base], idx_v)
        for j in range(NJ):
            pltpu.async_copy(
                tab_hbm.at[idx_v.at[pl.ds(j * IDXW, IDXW)]],
                rows_v.at[pl.ds(j * IDXW, IDXW)],
                gsem,
            )
        for j in range(NJ):
            pltpu.make_async_copy(
                tab_hbm.at[idx_v.at[pl.ds(j * IDXW, IDXW)]],
                rows_v.at[pl.ds(j * IDXW, IDXW)],
                gsem,
            ).wait()
        pltpu.sync_copy(rows_v, out_hbm.at[base])
        return ()

    lax.fori_loop(0, NCHUNK, step, ())


def _gather(flat_idx, pad_tab):
    mesh = plsc.VectorSubcoreMesh(core_axis_name="c", subcore_axis_name="s")
    f = pl.kernel(
        _body,
        out_type=jax.ShapeDtypeStruct((TOTAL // CHUNK, CHUNK, 128), jnp.float32),
        mesh=mesh,
        scratch_types=[
            pltpu.VMEM((CHUNK,), jnp.int32),
            pltpu.VMEM((CHUNK, 128), jnp.float32),
            pltpu.SemaphoreType.DMA,
        ],
        compiler_params=pltpu.CompilerParams(
            use_tc_tiling_on_sc=False, needs_layout_passes=False
        ),
    )
    return f(flat_idx, pad_tab)


def kernel(tuples, tables):
    offs = (jnp.arange(A, dtype=jnp.int32) * V)[None, :]
    flat_idx = (tuples + offs).reshape(TOTAL // CHUNK, CHUNK)
    pad_tab = jnp.pad(tables, ((0, 0), (0, 0), (0, D))).reshape(A * V, 2 * D)
    out = _gather(flat_idx, pad_tab)
    return out.reshape(TOTAL, 2 * D)[:, :D].reshape(B, A * D)


# speedup vs baseline: 1.7877x; 1.0850x over previous
import jax
import jax.numpy as jnp
from jax import lax
from jax.experimental import pallas as pl
from jax.experimental.pallas import tpu as pltpu
from jax.experimental.pallas import tpu_sc as plsc

A = 26
V = 100000
D = 64
B = 16384
TOTAL = B * A

NC, NS = 2, 16
NW = NC * NS

IDXW = 128
CHUNK = 512
NJ = CHUNK // IDXW
NCHUNK = TOTAL // NW // CHUNK

VB = 400
NBLK = A * (V // VB)
BPW = -(-NBLK // NW)


def _gather_body(idx_hbm, tab_hbm, out_hbm, idx_v, rows_v, gsem):
    wid = lax.axis_index("s") * NC + lax.axis_index("c")

    def step(c, _):
        base = wid * NCHUNK + c
        pltpu.sync_copy(idx_hbm.at[base], idx_v)
        for j in range(NJ):
            pltpu.async_copy(
                tab_hbm.at[idx_v.at[pl.ds(j * IDXW, IDXW)]],
                rows_v.at[pl.ds(j * IDXW, IDXW)],
                gsem,
            )
        for j in range(NJ):
            pltpu.make_async_copy(
                tab_hbm.at[idx_v.at[pl.ds(j * IDXW, IDXW)]],
                rows_v.at[pl.ds(j * IDXW, IDXW)],
                gsem,
            ).wait()
        pltpu.sync_copy(rows_v, out_hbm.at[base])
        return ()

    lax.fori_loop(0, NCHUNK, step, ())


def _gather(flat_idx, unit_tab):
    mesh = plsc.VectorSubcoreMesh(core_axis_name="c", subcore_axis_name="s")
    f = pl.kernel(
        _gather_body,
        out_type=jax.ShapeDtypeStruct((TOTAL // CHUNK, CHUNK, D), jnp.float32),
        mesh=mesh,
        scratch_types=[
            pltpu.VMEM((CHUNK,), jnp.int32),
            pltpu.VMEM((CHUNK, D), jnp.float32),
            pltpu.SemaphoreType.DMA,
        ],
        compiler_params=pltpu.CompilerParams(
            use_tc_tiling_on_sc=False, needs_layout_passes=False
        ),
    )
    return f(flat_idx, unit_tab)


def kernel(tuples, tables):
    offs = (jnp.arange(A, dtype=jnp.int32) * (2 * V))[None, :]
    flat_idx = (2 * tuples + offs).reshape(TOTAL // CHUNK, CHUNK)
    unit_tab = jnp.pad(tables, ((0, 0), (0, 0), (0, D))).reshape(2 * A * V, D)
    out = _gather(flat_idx, unit_tab)
    return out.reshape(B, A * D)


# speedup vs baseline: 1.9369x; 1.0835x over previous
import jax
import jax.numpy as jnp
from jax import lax
from jax.experimental import pallas as pl
from jax.experimental.pallas import tpu as pltpu
from jax.experimental.pallas import tpu_sc as plsc

A = 26
V = 100000
D = 64
B = 16384
TOTAL = B * A

NC, NS = 2, 16
NW = NC * NS

IDXW = 128
CHUNK = 512
NJ = CHUNK // IDXW
NCHUNK = TOTAL // NW // CHUNK


def _gather_body(idx_hbm, didx_hbm, tab_hbm, out_hbm, idx_v, didx_v, rows_v,
                 gsem, wsem):
    wid = lax.axis_index("s") * NC + lax.axis_index("c")

    def stage(c, s):
        base = wid * NCHUNK + c
        pltpu.sync_copy(idx_hbm.at[base], idx_v.at[s])
        pltpu.sync_copy(didx_hbm.at[base], didx_v.at[s])
        for j in range(NJ):
            pltpu.async_copy(
                tab_hbm.at[idx_v.at[s, pl.ds(j * IDXW, IDXW)]],
                rows_v.at[s, pl.ds(j * IDXW, IDXW)],
                gsem,
            )

    def wait_gathers(s):
        for j in range(NJ):
            pltpu.make_async_copy(
                tab_hbm.at[idx_v.at[s, pl.ds(j * IDXW, IDXW)]],
                rows_v.at[s, pl.ds(j * IDXW, IDXW)],
                gsem,
            ).wait()

    def fire_writes(s):
        for j in range(NJ):
            pltpu.async_copy(
                rows_v.at[s, pl.ds(j * IDXW, IDXW)],
                out_hbm.at[didx_v.at[s, j]],
                wsem,
            )

    def wait_writes(s):
        for j in range(NJ):
            pltpu.make_async_copy(
                rows_v.at[s, pl.ds(j * IDXW, IDXW)],
                out_hbm.at[didx_v.at[s, j]],
                wsem,
            ).wait()

    stage(0, 0)

    def step(i, _):
        for s in range(2):
            c = 2 * i + s

            @pl.when(c + 1 < NCHUNK)
            def _():
                @pl.when(c >= 1)
                def _():
                    wait_writes(1 - s)

                stage(c + 1, 1 - s)

            wait_gathers(s)
            fire_writes(s)
        return ()

    lax.fori_loop(0, NCHUNK // 2, step, ())
    wait_writes(1 - (NCHUNK - 1) % 2)
    wait_writes((NCHUNK - 1) % 2)


def _gather(flat_idx, dst_idx, unit_tab):
    mesh = plsc.VectorSubcoreMesh(core_axis_name="c", subcore_axis_name="s")
    f = pl.kernel(
        _gather_body,
        out_type=jax.ShapeDtypeStruct((TOTAL, D), jnp.float32),
        mesh=mesh,
        scratch_types=[
            pltpu.VMEM((2, CHUNK), jnp.int32),
            pltpu.VMEM((2, NJ, IDXW), jnp.int32),
            pltpu.VMEM((2, CHUNK, D), jnp.float32),
            pltpu.SemaphoreType.DMA,
            pltpu.SemaphoreType.DMA,
        ],
        compiler_params=pltpu.CompilerParams(
            use_tc_tiling_on_sc=False, needs_layout_passes=False
        ),
    )
    return f(flat_idx, dst_idx, unit_tab)


def kernel(tuples, tables):
    offs = (jnp.arange(A, dtype=jnp.int32) * (2 * V))[None, :]
    flat_idx = (2 * tuples + offs).reshape(TOTAL // CHUNK, CHUNK)
    unit_tab = jnp.pad(tables, ((0, 0), (0, 0), (0, D))).reshape(2 * A * V, D)
    r = jnp.arange(TOTAL, dtype=jnp.int32)
    b, a = r // A, r % A
    dst = (b >> 3) * (16 * (A // 2)) + (a >> 1) * 16 + (b & 7) * 2 + (a & 1)
    dst_idx = dst.reshape(TOTAL // CHUNK, NJ, IDXW)
    out = _gather(flat_idx, dst_idx, unit_tab)
    y = out.reshape(B // 8, A // 2, 8, 2 * D)
    return y.transpose(0, 2, 1, 3).reshape(B, A * D)
